# Initial kernel scaffold; baseline (speedup 1.0000x reference)
import jax
import jax.numpy as jnp
from jax.experimental import pallas as pl


def kernel(x, offsets, weight):
    raise NotImplementedError("write your pallas kernel here")


# speedup vs baseline: 43.1301x; 43.1301x over previous
import functools

import jax
import jax.numpy as jnp
from jax import lax
from jax.experimental import pallas as pl
from jax.experimental.pallas import tpu as pltpu
from jax.experimental.pallas import tpu_sc as plsc

VOCAB = 1000000
DIM = 64
B = 4096
L = 200
NBAGS = 20

PADL = 208
CH = 104
NW = 32
RPW = B // NW
ACC_ROWS = 24


def _seg_body(off_ref, seg_ref):
    blk = seg_ref.shape[0]
    i = pl.program_id(0)
    pos = lax.broadcasted_iota(jnp.int32, (blk, PADL), 1)
    seg = jnp.full((blk, PADL), -1, jnp.int32)
    for j in range(NBAGS):
        oj = off_ref[:, j][:, None]
        seg = seg + (oj <= pos).astype(jnp.int32)
    seg = jnp.where(pos >= L, NBAGS, seg)
    row = i * blk + lax.broadcasted_iota(jnp.int32, (blk, PADL), 0)
    sid = (row // RPW) // 2
    seg_ref[...] = seg + sid * ACC_ROWS


def _compute_seg(offsets):
    blk = 512
    return pl.pallas_call(
        _seg_body,
        grid=(B // blk,),
        in_specs=[pl.BlockSpec((blk, NBAGS), lambda i: (i, 0))],
        out_specs=pl.BlockSpec((blk, PADL), lambda i: (i, 0)),
        out_shape=jax.ShapeDtypeStruct((B, PADL), jnp.int32),
    )(offsets)


def _sc_body(x_hbm, seg_hbm, off_hbm, w_hbm, out_hbm,
             idx_v, seg_v, off_v, rows_v, acc_v, zero_v, shared_acc,
             sem):
    cid = lax.axis_index("c")
    sid = lax.axis_index("s")
    wid = sid * 2 + cid
    base = wid * RPW
    slot0 = sid * ACC_ROWS

    zero16 = jnp.zeros((16,), jnp.float32)
    for b in range(NBAGS + 1):
        for col in range(DIM // 16):
            zero_v[b, pl.ds(col * 16, 16)] = zero16

    def body(i, carry):
        row = base + i
        pltpu.sync_copy(x_hbm.at[row], idx_v)
        pltpu.sync_copy(seg_hbm.at[row], seg_v)
        pltpu.sync_copy(off_hbm.at[row], off_v)

        cp0 = pltpu.async_copy(w_hbm.at[idx_v.at[0]], rows_v.at[pl.ds(0, CH)], sem)
        cp1 = pltpu.async_copy(w_hbm.at[idx_v.at[1]], rows_v.at[pl.ds(CH, CH)], sem)

        pltpu.sync_copy(zero_v, shared_acc.at[pl.ds(slot0, NBAGS + 1)])
        c0 = off_v[pl.ds(1, 16)] - off_v[pl.ds(0, 16)]
        c1 = off_v[pl.ds(17, 16)] - off_v[pl.ds(16, 16)]
        r0 = 1.0 / jnp.maximum(c0.astype(jnp.float32), 1.0)
        r1 = 1.0 / jnp.maximum(c1.astype(jnp.float32), 1.0)

        cp0.wait()
        cp1.wait()

        pltpu.sync_copy(rows_v.at[pl.ds(0, CH)], shared_acc.at[seg_v.at[0]], add=True)
        pltpu.sync_copy(rows_v.at[pl.ds(CH, CH)], shared_acc.at[seg_v.at[1]], add=True)
        pltpu.sync_copy(shared_acc.at[pl.ds(slot0, NBAGS)], acc_v)

        for b in range(NBAGS):
            r = r0[b] if b < 16 else r1[b - 16]
            for col in range(DIM // 16):
                sl = pl.ds(col * 16, 16)
                acc_v[b, sl] = acc_v[b, sl] * r

        pltpu.sync_copy(acc_v, out_hbm.at[row])
        return carry

    lax.fori_loop(0, RPW, body, 0)


def _sc_call(x_pad, seg, off_pad, weight):
    mesh = plsc.VectorSubcoreMesh(core_axis_name="c", subcore_axis_name="s")
    f = pl.kernel(
        _sc_body,
        out_type=jax.ShapeDtypeStruct((B, NBAGS, DIM), jnp.float32),
        mesh=mesh,
        scratch_types=[
            pltpu.VMEM((2, CH), jnp.int32),
            pltpu.VMEM((2, CH), jnp.int32),
            pltpu.VMEM((40,), jnp.int32),
            pltpu.VMEM((PADL, DIM), jnp.float32),
            pltpu.VMEM((NBAGS, DIM), jnp.float32),
            pltpu.VMEM((NBAGS + 1, DIM), jnp.float32),
            pltpu.VMEM_SHARED((16 * ACC_ROWS, DIM), jnp.float32),
            pltpu.SemaphoreType.DMA,
        ],
        compiler_params=pltpu.CompilerParams(use_tc_tiling_on_sc=False),
    )
    return f(x_pad, seg, off_pad, weight)


def kernel(x, offsets, weight):
    x_pad = jnp.concatenate(
        [x, jnp.zeros((B, PADL - L), jnp.int32)], axis=1
    ).reshape(B, 2, CH)
    off_pad = jnp.concatenate(
        [offsets, jnp.full((B, 40 - NBAGS), L, jnp.int32)], axis=1
    )
    seg = _compute_seg(offsets).reshape(B, 2, CH)
    return _sc_call(x_pad, seg, off_pad, weight)
